# Initial kernel scaffold; baseline (speedup 1.0000x reference)
import jax
import jax.numpy as jnp
from jax.experimental import pallas as pl


def kernel(F0, vertices0, k):
    raise NotImplementedError("write your pallas kernel here")


# speedup vs baseline: 1.0452x; 1.0452x over previous
import jax
import jax.numpy as jnp
from jax.experimental import pallas as pl

N = 64
S = 16
C = 2048


def _kern(f_ref, v_ref, fout_ref, vout_ref):
    F = f_ref[0]
    V = v_ref[0]

    iota_vc = jax.lax.broadcasted_iota(jnp.int32, (N, C), 0)
    M = jnp.max(F, axis=0, keepdims=True)
    amax = jnp.min(jnp.where(F == M, iota_vc, N), axis=0, keepdims=True)

    eq = (amax == iota_vc).astype(jnp.float32)
    counts = jnp.sum(eq, axis=1, keepdims=True).astype(jnp.int32)

    present = counts > 0
    U = jnp.sum(present.astype(jnp.int32))

    v_i = jax.lax.broadcasted_iota(jnp.int32, (N, N), 0)
    w_i = jax.lax.broadcasted_iota(jnp.int32, (N, N), 1)
    counts_w = jnp.transpose(counts)
    present_w = counts_w > 0

    rank_w = jnp.sum(
        jnp.where(present & (v_i < w_i), 1, 0), axis=0, keepdims=True
    )

    lane_id = jax.lax.broadcasted_iota(jnp.int32, (1, N), 1)
    sub_id = jax.lax.broadcasted_iota(jnp.int32, (N, 1), 0)
    key_w = counts_w * N + (N - 1 - lane_id)
    key_v = counts * N + (N - 1 - sub_id)
    R_w = jnp.sum(jnp.where(key_v > key_w, 1, 0), axis=0, keepdims=True)

    j_i = jax.lax.broadcasted_iota(jnp.int32, (S, N), 0)
    w2_i = jax.lax.broadcasted_iota(jnp.int32, (S, N), 1)
    pad = S - U
    rtarget = jnp.maximum(j_i - pad, 0)
    m_pad = present_w & (rank_w == rtarget)
    out_pad = jnp.sum(jnp.where(m_pad, w2_i, 0), axis=1, keepdims=True)
    m_top = R_w == j_i
    out_top = jnp.sum(jnp.where(m_top, w2_i, 0), axis=1, keepdims=True)
    idx = jnp.where(U < S, out_pad, out_top)

    onehot = (idx == w2_i).astype(jnp.float32)
    fout_ref[0] = jnp.dot(onehot, F, preferred_element_type=jnp.float32)
    vout_ref[0] = jnp.dot(onehot, V, preferred_element_type=jnp.float32)


def kernel(F0, vertices0, k):
    B = F0.shape[0]
    out_shapes = (
        jax.ShapeDtypeStruct((B, S, C), jnp.float32),
        jax.ShapeDtypeStruct((B, S, 3), jnp.float32),
    )
    F_new, vertices_new = pl.pallas_call(
        _kern,
        grid=(B,),
        in_specs=[
            pl.BlockSpec((1, N, C), lambda b: (b, 0, 0)),
            pl.BlockSpec((1, N, 3), lambda b: (b, 0, 0)),
        ],
        out_specs=(
            pl.BlockSpec((1, S, C), lambda b: (b, 0, 0)),
            pl.BlockSpec((1, S, 3), lambda b: (b, 0, 0)),
        ),
        out_shape=out_shapes,
    )(F0, vertices0)
    return (F_new, vertices_new)


# speedup vs baseline: 1.6760x; 1.6035x over previous
import jax
import jax.numpy as jnp
from jax.experimental import pallas as pl

N = 64
S = 16
C = 2048
BB = 8


def _one_sample(F, V, fout_ref, vout_ref, i):
    iota_vc = jax.lax.broadcasted_iota(jnp.int32, (N, C), 0)
    M = jnp.max(F, axis=0, keepdims=True)
    amax = jnp.min(jnp.where(F == M, iota_vc, N), axis=0, keepdims=True)

    eq = (amax == iota_vc).astype(jnp.float32)
    counts_f = jnp.dot(eq, jnp.ones((C, 1), jnp.float32),
                       preferred_element_type=jnp.float32)
    counts = counts_f.astype(jnp.int32)

    present = counts > 0
    U = jnp.sum(present.astype(jnp.int32))

    v_i = jax.lax.broadcasted_iota(jnp.int32, (N, N), 0)
    w_i = jax.lax.broadcasted_iota(jnp.int32, (N, N), 1)
    counts_w = jnp.transpose(counts)
    present_w = counts_w > 0

    rank_w = jnp.sum(
        jnp.where(present & (v_i < w_i), 1, 0), axis=0, keepdims=True
    )

    lane_id = jax.lax.broadcasted_iota(jnp.int32, (1, N), 1)
    sub_id = jax.lax.broadcasted_iota(jnp.int32, (N, 1), 0)
    key_w = counts_w * N + (N - 1 - lane_id)
    key_v = counts * N + (N - 1 - sub_id)
    R_w = jnp.sum(jnp.where(key_v > key_w, 1, 0), axis=0, keepdims=True)

    j_i = jax.lax.broadcasted_iota(jnp.int32, (S, N), 0)
    w2_i = jax.lax.broadcasted_iota(jnp.int32, (S, N), 1)
    pad = S - U
    rtarget = jnp.maximum(j_i - pad, 0)
    m_pad = present_w & (rank_w == rtarget)
    out_pad = jnp.sum(jnp.where(m_pad, w2_i, 0), axis=1, keepdims=True)
    m_top = R_w == j_i
    out_top = jnp.sum(jnp.where(m_top, w2_i, 0), axis=1, keepdims=True)
    idx = jnp.where(U < S, out_pad, out_top)

    onehot = (idx == w2_i).astype(jnp.float32)
    fout_ref[i] = jnp.dot(onehot, F, preferred_element_type=jnp.float32)
    vout_ref[i] = jnp.dot(onehot, V, preferred_element_type=jnp.float32)


def _kern(f_ref, v_ref, fout_ref, vout_ref):
    for i in range(BB):
        _one_sample(f_ref[i], v_ref[i], fout_ref, vout_ref, i)


def kernel(F0, vertices0, k):
    B = F0.shape[0]
    out_shapes = (
        jax.ShapeDtypeStruct((B, S, C), jnp.float32),
        jax.ShapeDtypeStruct((B, S, 3), jnp.float32),
    )
    F_new, vertices_new = pl.pallas_call(
        _kern,
        grid=(B // BB,),
        in_specs=[
            pl.BlockSpec((BB, N, C), lambda b: (b, 0, 0)),
            pl.BlockSpec((BB, N, 3), lambda b: (b, 0, 0)),
        ],
        out_specs=(
            pl.BlockSpec((BB, S, C), lambda b: (b, 0, 0)),
            pl.BlockSpec((BB, S, 3), lambda b: (b, 0, 0)),
        ),
        out_shape=out_shapes,
    )(F0, vertices0)
    return (F_new, vertices_new)
